# baseline (speedup 1.0000x reference)
import functools

import jax
import jax.numpy as jnp
from jax import lax
from jax.experimental import pallas as pl
from jax.experimental.pallas import tpu as pltpu
from jax.experimental.pallas import tpu_sc as plsc

_GAMMA = 12.0
_D = 64
_L = 16
_NPH = 128
_RING = 2


def _pack_body(top_ref, bot_ref, o_ref):
    o_ref[...] = jnp.concatenate([top_ref[...], bot_ref[...]], axis=1)


@functools.lru_cache(maxsize=None)
def _build_pack(NE):
    H = NE // 2
    BLK = 4000
    G = H // BLK
    return pl.pallas_call(
        _pack_body,
        grid=(G,),
        in_specs=[
            pl.BlockSpec((BLK, _D), lambda i: (i, 0)),
            pl.BlockSpec((BLK, _D), lambda i: (i + G, 0)),
        ],
        out_specs=pl.BlockSpec((BLK, 2 * _D), lambda i: (i, 0)),
        out_shape=jax.ShapeDtypeStruct((H, 2 * _D), jnp.float32),
    )


@functools.lru_cache(maxsize=None)
def _build_sc_kernel(B, NNEG, THALF, XHALF):
    info = plsc.get_sparse_core_info()
    NC, NS = info.num_cores, info.num_subcores
    NW = NC * NS
    QW = B // NW
    HROWS = B * NNEG // _NPH
    HW = HROWS // NW

    mesh = plsc.VectorSubcoreMesh(core_axis_name="c", subcore_axis_name="s")

    def body(x_hbm, tab_hbm, tgt_hbm, tgtrow_hbm, pos_hbm, posrow_hbm,
             nidx_hbm, nrow_hbm,
             plog_hbm, nlog_hbm,
             tgt_v, pos_v, tgtrow_v, posrow_v, nrow_v, norig_v,
             tmp128_v, pred_v, nbuf_v, plog_v, nlogbuf_v,
             sem_a, sem_n0, sem_n1, sem_o0, sem_o1, sem_w0, sem_w1):
        wid = lax.axis_index("s") * NC + lax.axis_index("c")
        qbase = wid * QW
        hbase = wid * HW
        iota = lax.iota(jnp.int32, _L)

        pltpu.sync_copy(tgt_hbm.at[pl.ds(qbase, QW)], tgt_v)
        pltpu.sync_copy(pos_hbm.at[pl.ds(qbase, QW)], pos_v)
        pltpu.sync_copy(tgtrow_hbm.at[pl.ds(qbase, QW)], tgtrow_v)
        pltpu.sync_copy(posrow_hbm.at[pl.ds(qbase, QW)], posrow_v)
        pltpu.sync_copy(nrow_hbm.at[pl.ds(hbase, HW)], nrow_v)

        bufs = [nbuf_v.at[k] for k in range(_RING)]
        sems = [sem_n0, sem_n1]
        osems = [sem_o0, sem_o1]

        def start_h(h, par):
            pltpu.make_async_copy(
                tab_hbm.at[nrow_v.at[h]], bufs[par], sems[par]).start()
            pltpu.make_async_copy(
                nidx_hbm.at[hbase + h], norig_v.at[par], osems[par]).start()

        pltpu.async_copy(x_hbm.at[tgtrow_v], tmp128_v, sem_a).wait()

        for k in range(_RING):
            start_h(k, k)

        def compact(i, carry):
            g = i // _L
            jj = i % _L
            t_vec = tgt_v[pl.ds(g * _L, _L)]
            t_i = jnp.sum(jnp.where(iota == jj, t_vec, 0))
            sel = (jnp.zeros((_L,), jnp.int32) + t_i) >= XHALF
            for c in range(4):
                pred_v[i, pl.ds(c * _L, _L)] = jnp.where(
                    sel,
                    tmp128_v[i, pl.ds(_D + c * _L, _L)],
                    tmp128_v[i, pl.ds(c * _L, _L)])
            return carry
        lax.fori_loop(0, QW, compact, 0)

        def l1_row(ref, j, chunks, base):
            parts = [jnp.abs(ref[j, pl.ds(base + c * _L, _L)] - chunks[c])
                     for c in range(4)]
            v = (parts[0] + parts[1]) + (parts[2] + parts[3])
            return jnp.sum(v)

        pltpu.async_copy(tab_hbm.at[posrow_v], tmp128_v, sem_a).wait()

        def pos_group(qg, carry):
            out_lo = jnp.zeros((_L,), jnp.float32)
            out_hi = jnp.zeros((_L,), jnp.float32)
            for jj in range(_L):
                i = qg * _L + jj
                chunks = [pred_v[i, pl.ds(c * _L, _L)] for c in range(4)]
                s_lo = l1_row(tmp128_v, i, chunks, 0)
                s_hi = l1_row(tmp128_v, i, chunks, _D)
                out_lo = jnp.where(iota == jj, s_lo, out_lo)
                out_hi = jnp.where(iota == jj, s_hi, out_hi)
            p_vec = pos_v[pl.ds(qg * _L, _L)]
            out = jnp.where(p_vec >= THALF, out_hi, out_lo)
            plog_v[pl.ds(qg * _L, _L)] = _GAMMA - out
            return carry
        lax.fori_loop(0, QW // _L, pos_group, 0)

        wsems = [sem_w0, sem_w1]

        def compute_h(h, par, ws):
            q = h // 2
            pltpu.make_async_copy(
                tab_hbm.at[nrow_v.at[h]], bufs[par], sems[par]).wait()
            pltpu.make_async_copy(
                nidx_hbm.at[hbase + h], norig_v.at[par], osems[par]).wait()
            chunks = [pred_v[q, pl.ds(c * _L, _L)] for c in range(4)]

            @pl.when(h >= 2)
            def _():
                pltpu.make_async_copy(
                    nlogbuf_v.at[ws],
                    nlog_hbm.at[h - 2 + hbase], wsems[ws]).wait()

            def neg_group(g, c2):
                out_lo = jnp.zeros((_L,), jnp.float32)
                out_hi = jnp.zeros((_L,), jnp.float32)
                for jj in range(_L):
                    j = g * _L + jj
                    s_lo = l1_row(bufs[par], j, chunks, 0)
                    s_hi = l1_row(bufs[par], j, chunks, _D)
                    out_lo = jnp.where(iota == jj, s_lo, out_lo)
                    out_hi = jnp.where(iota == jj, s_hi, out_hi)
                n_vec = norig_v[par, pl.ds(g * _L, _L)]
                out = jnp.where(n_vec >= THALF, out_hi, out_lo)
                nlogbuf_v[ws, pl.ds(g * _L, _L)] = _GAMMA - out
                return c2
            lax.fori_loop(0, _NPH // _L, neg_group, 0)
            pltpu.make_async_copy(
                nlogbuf_v.at[ws], nlog_hbm.at[h + hbase], wsems[ws]).start()

        def neg_pair(hh, carry):
            h = hh * _RING
            for k in range(_RING):
                compute_h(h + k, k, k % 2)
                @pl.when(h + k + _RING < HW)
                def _():
                    start_h(h + k + _RING, k)
            return carry
        lax.fori_loop(0, HW // _RING, neg_pair, 0)

        for ws in range(2):
            pltpu.make_async_copy(
                nlogbuf_v.at[ws],
                nlog_hbm.at[HW - 2 + ws + hbase], wsems[ws]).wait()

        pltpu.sync_copy(plog_v, plog_hbm.at[pl.ds(qbase, QW)])

    return pl.kernel(
        body,
        mesh=mesh,
        compiler_params=pltpu.CompilerParams(
            needs_layout_passes=False, use_tc_tiling_on_sc=True),
        out_type=[
            jax.ShapeDtypeStruct((B,), jnp.float32),
            jax.ShapeDtypeStruct((HROWS, _NPH), jnp.float32),
        ],
        scratch_types=[
            pltpu.VMEM((QW,), jnp.int32),
            pltpu.VMEM((QW,), jnp.int32),
            pltpu.VMEM((QW,), jnp.int32),
            pltpu.VMEM((QW,), jnp.int32),
            pltpu.VMEM((HW, _NPH), jnp.int32),
            pltpu.VMEM((_RING, _NPH), jnp.int32),
            pltpu.VMEM((QW, 2 * _D), jnp.float32),
            pltpu.VMEM((QW, _D), jnp.float32),
            pltpu.VMEM((_RING, _NPH, 2 * _D), jnp.float32),
            pltpu.VMEM((QW,), jnp.float32),
            pltpu.VMEM((2, _NPH), jnp.float32),
            pltpu.SemaphoreType.DMA,
            pltpu.SemaphoreType.DMA,
            pltpu.SemaphoreType.DMA,
            pltpu.SemaphoreType.DMA,
            pltpu.SemaphoreType.DMA,
            pltpu.SemaphoreType.DMA,
            pltpu.SemaphoreType.DMA,
        ],
    )


def kernel(x, entity_table, target_node_idxes, positive_samples, negative_samples):
    B, NNEG = negative_samples.shape
    NE = entity_table.shape[0]
    NX = x.shape[0]
    TH = NE // 2
    XH = NX // 2
    tgt = target_node_idxes.astype(jnp.int32)
    pos = positive_samples.astype(jnp.int32)
    nidx = negative_samples.astype(jnp.int32).reshape(B * NNEG // _NPH, _NPH)
    tgtrow = jnp.where(tgt >= XH, tgt - XH, tgt)
    posrow = jnp.where(pos >= TH, pos - TH, pos)
    nrow = jnp.where(nidx >= TH, nidx - TH, nidx)
    xp = jnp.concatenate([x[:XH], x[XH:]], axis=1)
    tabp = _build_pack(NE)(entity_table, entity_table)
    sc = _build_sc_kernel(B, NNEG, TH, XH)
    plog, nlog = sc(xp, tabp, tgt, tgtrow, pos, posrow, nidx, nrow)
    return plog.reshape(B, 1), nlog.reshape(B, NNEG)
